# baseline (speedup 1.0000x reference)
import jax
import jax.numpy as jnp
from jax import lax
from jax.experimental import pallas as pl
from jax.experimental.pallas import tpu as pltpu

B = 16384
N = 3
F = 128
EPS = 1e-5
TB = 1024
T = B // TB


def _gcn_kernel(h_hbm, wt_ref, b_ref, g3_ref, be3_ref, a_ref, out_hbm,
                hbuf, obuf, x_store, sum_ref, sq_ref, in_sems, out_sems):
    p = pl.program_id(0)
    t = pl.program_id(1)

    def in_copy(tt, slot):
        return pltpu.make_async_copy(
            h_hbm.at[pl.ds(tt * TB, TB)],
            hbuf.at[slot],
            in_sems.at[slot],
        )

    def out_copy(tt, slot):
        return pltpu.make_async_copy(
            obuf.at[slot],
            out_hbm.at[pl.ds(tt * TB, TB)],
            out_sems.at[slot],
        )

    @pl.when(p == 0)
    def _phase0():
        slot = lax.rem(t, 2)

        @pl.when(t == 0)
        def _():
            in_copy(0, 0).start()

        @pl.when(t + 1 < T)
        def _():
            in_copy(t + 1, lax.rem(t + 1, 2)).start()

        in_copy(t, slot).wait()

        wt = wt_ref[...]
        xs = [jnp.dot(hbuf[slot, :, n, :], wt,
                      preferred_element_type=jnp.float32)
              + b_ref[...] for n in range(N)]
        x = jnp.concatenate(xs, axis=1)
        x_store[pl.ds(t * TB, TB), :] = x
        part_s = jnp.sum(x, axis=0, keepdims=True)
        part_q = jnp.sum(x * x, axis=0, keepdims=True)

        @pl.when(t == 0)
        def _():
            sum_ref[...] = part_s
            sq_ref[...] = part_q

        @pl.when(t != 0)
        def _():
            sum_ref[...] = sum_ref[...] + part_s
            sq_ref[...] = sq_ref[...] + part_q

    @pl.when(p == 1)
    def _phase1():
        slot = lax.rem(t, 2)

        @pl.when(t >= 2)
        def _():
            out_copy(t - 2, slot).wait()

        cnt = float(B * F)
        s = sum_ref[...]
        q = sq_ref[...]
        m0 = jnp.sum(s[:, 0:F]) / cnt
        m1 = jnp.sum(s[:, F:2 * F]) / cnt
        m2 = jnp.sum(s[:, 2 * F:3 * F]) / cnt
        v0 = jnp.sum(q[:, 0:F]) / cnt - m0 * m0
        v1 = jnp.sum(q[:, F:2 * F]) / cnt - m1 * m1
        v2 = jnp.sum(q[:, 2 * F:3 * F]) / cnt - m2 * m2
        lane = jax.lax.broadcasted_iota(jnp.int32, (1, 3 * F), 1) // F
        mean_vec = jnp.where(lane == 0, m0, jnp.where(lane == 1, m1, m2))
        var_vec = jnp.where(lane == 0, v0, jnp.where(lane == 1, v1, v2))
        scale_vec = g3_ref[...] * jax.lax.rsqrt(var_vec + EPS)
        shift_vec = be3_ref[...] - mean_vec * scale_vec

        x = x_store[pl.ds(t * TB, TB), :]
        y = x * scale_vec + shift_vec
        ys = [y[:, n * F:(n + 1) * F] for n in range(N)]
        for d in range(N):
            o = (a_ref[d, 0] * ys[0] + a_ref[d, 1] * ys[1]
                 + a_ref[d, 2] * ys[2])
            obuf[slot, :, d, :] = jnp.maximum(o, 0.0)
        out_copy(t, slot).start()

        @pl.when(t == T - 1)
        def _():
            out_copy(t - 1, lax.rem(t - 1, 2)).wait()
            out_copy(t, slot).wait()


def kernel(h, W, b, gamma, beta, src, dst):
    wt = W.T
    b1 = b[None, :]
    g3 = jnp.repeat(gamma, F)[None, :]
    be3 = jnp.repeat(beta, F)[None, :]
    nodes = jnp.arange(N, dtype=src.dtype)
    adj = jnp.sum(
        (dst[:, None, None] == nodes[None, :, None])
        & (src[:, None, None] == nodes[None, None, :]),
        axis=0,
    ).astype(jnp.float32)

    return pl.pallas_call(
        _gcn_kernel,
        grid=(2, T),
        in_specs=[
            pl.BlockSpec(memory_space=pltpu.MemorySpace.HBM),
            pl.BlockSpec((F, F), lambda p, t: (0, 0)),
            pl.BlockSpec((1, F), lambda p, t: (0, 0)),
            pl.BlockSpec((1, N * F), lambda p, t: (0, 0)),
            pl.BlockSpec((1, N * F), lambda p, t: (0, 0)),
            pl.BlockSpec(memory_space=pltpu.SMEM),
        ],
        out_specs=pl.BlockSpec(memory_space=pltpu.MemorySpace.HBM),
        out_shape=jax.ShapeDtypeStruct((B, N, F), jnp.float32),
        scratch_shapes=[
            pltpu.VMEM((2, TB, N, F), jnp.float32),
            pltpu.VMEM((2, TB, N, F), jnp.float32),
            pltpu.VMEM((B, N * F), jnp.float32),
            pltpu.VMEM((1, N * F), jnp.float32),
            pltpu.VMEM((1, N * F), jnp.float32),
            pltpu.SemaphoreType.DMA((2,)),
            pltpu.SemaphoreType.DMA((2,)),
        ],
        compiler_params=pltpu.CompilerParams(
            dimension_semantics=("arbitrary", "arbitrary"),
            vmem_limit_bytes=60 * 1024 * 1024,
        ),
    )(h, wt, b1, g3, be3, adj)


# speedup vs baseline: 1.3160x; 1.3160x over previous
import jax
import jax.numpy as jnp
from jax import lax
from jax.experimental import pallas as pl
from jax.experimental.pallas import tpu as pltpu

B = 16384
N = 3
F = 128
EPS = 1e-5
TB = 4096
T = B // TB


def _gcn_kernel(h_hbm, wt_ref, b_ref, g3_ref, be3_ref, a_ref, out_hbm,
                hbuf, obuf, x_store, sum_ref, sq_ref, in_sems, out_sems):
    p = pl.program_id(0)
    t = pl.program_id(1)

    def in_copy(tt, slot, n):
        return pltpu.make_async_copy(
            h_hbm.at[pl.ds(tt * TB, TB), n, :],
            hbuf.at[slot, n],
            in_sems.at[slot, n],
        )

    def out_copy(tt, slot, n):
        return pltpu.make_async_copy(
            obuf.at[slot, n],
            out_hbm.at[pl.ds(tt * TB, TB), n, :],
            out_sems.at[slot, n],
        )

    @pl.when(p == 0)
    def _phase0():
        slot = lax.rem(t, 2)

        @pl.when(t == 0)
        def _():
            for n in range(N):
                in_copy(0, 0, n).start()

        @pl.when(t + 1 < T)
        def _():
            for n in range(N):
                in_copy(t + 1, lax.rem(t + 1, 2), n).start()

        for n in range(N):
            in_copy(t, slot, n).wait()

        wt = wt_ref[...]
        xs = [jnp.dot(hbuf[slot, n], wt, preferred_element_type=jnp.float32)
              + b_ref[...] for n in range(N)]
        x = jnp.concatenate(xs, axis=1)
        x_store[pl.ds(t * TB, TB), :] = x
        part_s = jnp.sum(x, axis=0, keepdims=True)
        part_q = jnp.sum(x * x, axis=0, keepdims=True)

        @pl.when(t == 0)
        def _():
            sum_ref[...] = part_s
            sq_ref[...] = part_q

        @pl.when(t != 0)
        def _():
            sum_ref[...] = sum_ref[...] + part_s
            sq_ref[...] = sq_ref[...] + part_q

    @pl.when(p == 1)
    def _phase1():
        slot = lax.rem(t, 2)

        @pl.when(t >= 2)
        def _():
            for n in range(N):
                out_copy(t - 2, slot, n).wait()

        cnt = float(B * F)
        s = sum_ref[...]
        q = sq_ref[...]
        m0 = jnp.sum(s[:, 0:F]) / cnt
        m1 = jnp.sum(s[:, F:2 * F]) / cnt
        m2 = jnp.sum(s[:, 2 * F:3 * F]) / cnt
        v0 = jnp.sum(q[:, 0:F]) / cnt - m0 * m0
        v1 = jnp.sum(q[:, F:2 * F]) / cnt - m1 * m1
        v2 = jnp.sum(q[:, 2 * F:3 * F]) / cnt - m2 * m2
        lane = jax.lax.broadcasted_iota(jnp.int32, (1, 3 * F), 1) // F
        mean_vec = jnp.where(lane == 0, m0, jnp.where(lane == 1, m1, m2))
        var_vec = jnp.where(lane == 0, v0, jnp.where(lane == 1, v1, v2))
        scale_vec = g3_ref[...] * jax.lax.rsqrt(var_vec + EPS)
        shift_vec = be3_ref[...] - mean_vec * scale_vec

        x = x_store[pl.ds(t * TB, TB), :]
        y = x * scale_vec + shift_vec
        ys = [y[:, n * F:(n + 1) * F] for n in range(N)]
        for d in range(N):
            o = (a_ref[d, 0] * ys[0] + a_ref[d, 1] * ys[1]
                 + a_ref[d, 2] * ys[2])
            obuf[slot, d] = jnp.maximum(o, 0.0)
        for n in range(N):
            out_copy(t, slot, n).start()

        @pl.when(t == T - 1)
        def _():
            for n in range(N):
                out_copy(t - 1, lax.rem(t - 1, 2), n).wait()
                out_copy(t, slot, n).wait()


def kernel(h, W, b, gamma, beta, src, dst):
    wt = W.T
    b1 = b[None, :]
    g3 = jnp.repeat(gamma, F)[None, :]
    be3 = jnp.repeat(beta, F)[None, :]
    nodes = jnp.arange(N, dtype=src.dtype)
    adj = jnp.sum(
        (dst[:, None, None] == nodes[None, :, None])
        & (src[:, None, None] == nodes[None, None, :]),
        axis=0,
    ).astype(jnp.float32)

    return pl.pallas_call(
        _gcn_kernel,
        grid=(2, T),
        in_specs=[
            pl.BlockSpec(memory_space=pltpu.MemorySpace.HBM),
            pl.BlockSpec((F, F), lambda p, t: (0, 0)),
            pl.BlockSpec((1, F), lambda p, t: (0, 0)),
            pl.BlockSpec((1, N * F), lambda p, t: (0, 0)),
            pl.BlockSpec((1, N * F), lambda p, t: (0, 0)),
            pl.BlockSpec(memory_space=pltpu.SMEM),
        ],
        out_specs=pl.BlockSpec(memory_space=pltpu.MemorySpace.HBM),
        out_shape=jax.ShapeDtypeStruct((B, N, F), jnp.float32),
        scratch_shapes=[
            pltpu.VMEM((2, N, TB, F), jnp.float32),
            pltpu.VMEM((2, N, TB, F), jnp.float32),
            pltpu.VMEM((B, N * F), jnp.float32),
            pltpu.VMEM((1, N * F), jnp.float32),
            pltpu.VMEM((1, N * F), jnp.float32),
            pltpu.SemaphoreType.DMA((2, N)),
            pltpu.SemaphoreType.DMA((2, N)),
        ],
        compiler_params=pltpu.CompilerParams(
            dimension_semantics=("arbitrary", "arbitrary"),
            vmem_limit_bytes=60 * 1024 * 1024,
        ),
    )(h, wt, b1, g3, be3, adj)
